# baseline (speedup 1.0000x reference)
import functools

import jax
from jax import lax
import jax.numpy as jnp
from jax.experimental import pallas as pl
from jax.experimental.pallas import tpu as pltpu
from jax.experimental.pallas import tpu_sc as plsc

_K = 200
_NW = 32
_NBUF = 3


def kernel(atomic_numbers, table):
    n = atomic_numbers.shape[0]
    dim = table.shape[1]
    m = n // _K
    niter = (m + _NW - 1) // _NW
    mesh = plsc.VectorSubcoreMesh(core_axis_name="c", subcore_axis_name="s")

    scratch = (
        [pltpu.VMEM((_K,), jnp.int32) for _ in range(_NBUF)]
        + [pltpu.VMEM((_K, dim), table.dtype) for _ in range(_NBUF)]
        + [pltpu.SemaphoreType.DMA for _ in range(3 * _NBUF)]
    )

    @functools.partial(
        pl.kernel,
        out_type=jax.ShapeDtypeStruct((m, _K, dim), table.dtype),
        mesh=mesh,
        scratch_types=scratch,
    )
    def gather_kernel(table_hbm, idx_hbm, out_hbm, *sc):
        idx_v = sc[0:_NBUF]
        row_v = sc[_NBUF:2 * _NBUF]
        sem_i = sc[2 * _NBUF:3 * _NBUF]
        sem_g = sc[3 * _NBUF:4 * _NBUF]
        sem_w = sc[4 * _NBUF:5 * _NBUF]

        wid = lax.axis_index("s") * 2 + lax.axis_index("c")

        def chunk(t):
            return wid + t * _NW

        def start_idx(t, b):
            pltpu.async_copy(idx_hbm.at[chunk(t)], idx_v[b], sem_i[b])

        def wait_idx(b):
            pltpu.make_async_copy(idx_hbm.at[0], idx_v[b], sem_i[b]).wait()

        def start_g(b):
            pltpu.async_copy(table_hbm.at[idx_v[b]], row_v[b], sem_g[b])

        def wait_g(b):
            pltpu.make_async_copy(
                table_hbm.at[idx_v[b]], row_v[b], sem_g[b]
            ).wait()

        def start_w(t, b):
            pltpu.async_copy(row_v[b], out_hbm.at[chunk(t)], sem_w[b])

        def wait_w(b):
            pltpu.make_async_copy(row_v[b], out_hbm.at[0], sem_w[b]).wait()

        for t in range(min(_NBUF, niter)):
            start_idx(t, t % _NBUF)

        for t in range(niter):
            b = t % _NBUF
            if t >= _NBUF:
                wait_w(b)
            if t == niter - 1:
                def last_a(b=b, t=t):
                    wait_idx(b)
                    start_g(b)
                pl.when(chunk(t) < m)(last_a)
            else:
                wait_idx(b)
                start_g(b)
            if t >= 1:
                u = t - 1
                pb = u % _NBUF
                wait_g(pb)
                start_w(u, pb)
                if u + _NBUF < niter:
                    tn = u + _NBUF
                    if tn == niter - 1:
                        pl.when(chunk(tn) < m)(
                            lambda tn=tn, pb=pb: start_idx(tn, pb)
                        )
                    else:
                        start_idx(tn, pb)

        lb = (niter - 1) % _NBUF

        def last_b(lb=lb):
            wait_g(lb)
            start_w(niter - 1, lb)
            wait_w(lb)

        pl.when(chunk(niter - 1) < m)(last_b)
        wait_w((niter - _NBUF) % _NBUF)
        wait_w((niter - 2) % _NBUF)

    idx2d = atomic_numbers.reshape(m, _K)
    return gather_kernel(table, idx2d).reshape(n, dim)


# speedup vs baseline: 1.6263x; 1.6263x over previous
import functools

import jax
from jax import lax
import jax.numpy as jnp
from jax.experimental import pallas as pl
from jax.experimental import pallas as pl_tc
from jax.experimental.pallas import tpu as pltpu
from jax.experimental.pallas import tpu_sc as plsc

_K = 200
_NW = 32
_NBUF = 3
_M_SC = 52
_B_TC = 800


def _sc_gather(idx2d, table):
    m = idx2d.shape[0]
    dim = table.shape[1]
    niter = (m + _NW - 1) // _NW
    mesh = plsc.VectorSubcoreMesh(core_axis_name="c", subcore_axis_name="s")

    scratch = (
        [pltpu.VMEM((_K,), jnp.int32) for _ in range(_NBUF)]
        + [pltpu.VMEM((_K, dim), table.dtype) for _ in range(_NBUF)]
        + [pltpu.SemaphoreType.DMA for _ in range(3 * _NBUF)]
    )

    @functools.partial(
        pl.kernel,
        out_type=jax.ShapeDtypeStruct((m, _K, dim), table.dtype),
        mesh=mesh,
        scratch_types=scratch,
    )
    def gather_kernel(table_hbm, idx_hbm, out_hbm, *sc):
        idx_v = sc[0:_NBUF]
        row_v = sc[_NBUF:2 * _NBUF]
        sem_i = sc[2 * _NBUF:3 * _NBUF]
        sem_g = sc[3 * _NBUF:4 * _NBUF]
        sem_w = sc[4 * _NBUF:5 * _NBUF]

        wid = lax.axis_index("s") * 2 + lax.axis_index("c")

        def chunk(t):
            return wid + t * _NW

        def start_idx(t, b):
            pltpu.async_copy(idx_hbm.at[chunk(t)], idx_v[b], sem_i[b])

        def wait_idx(b):
            pltpu.make_async_copy(idx_hbm.at[0], idx_v[b], sem_i[b]).wait()

        def start_g(b):
            pltpu.async_copy(table_hbm.at[idx_v[b]], row_v[b], sem_g[b])

        def wait_g(b):
            pltpu.make_async_copy(
                table_hbm.at[idx_v[b]], row_v[b], sem_g[b]
            ).wait()

        def start_w(t, b):
            pltpu.async_copy(row_v[b], out_hbm.at[chunk(t)], sem_w[b])

        def wait_w(b):
            pltpu.make_async_copy(row_v[b], out_hbm.at[0], sem_w[b]).wait()

        for t in range(min(_NBUF, niter)):
            if t == niter - 1:
                pl.when(chunk(t) < m)(lambda t=t: start_idx(t, t % _NBUF))
            else:
                start_idx(t, t % _NBUF)

        for t in range(niter):
            b = t % _NBUF
            if t >= _NBUF:
                wait_w(b)
            if t == niter - 1:
                def last_a(b=b, t=t):
                    wait_idx(b)
                    start_g(b)
                pl.when(chunk(t) < m)(last_a)
            else:
                wait_idx(b)
                start_g(b)
            if t >= 1:
                u = t - 1
                pb = u % _NBUF
                wait_g(pb)
                start_w(u, pb)
                if u + _NBUF < niter:
                    tn = u + _NBUF
                    if tn == niter - 1:
                        pl.when(chunk(tn) < m)(
                            lambda tn=tn, pb=pb: start_idx(tn, pb)
                        )
                    else:
                        start_idx(tn, pb)

        lb = (niter - 1) % _NBUF

        def last_b(lb=lb):
            wait_g(lb)
            start_w(niter - 1, lb)
            wait_w(lb)

        pl.when(chunk(niter - 1) < m)(last_b)
        for t in range(max(0, niter - _NBUF), niter - 1):
            wait_w(t % _NBUF)

    return gather_kernel(table, idx2d)


def _tc_onehot_body(idx_ref, tab_ref, out_ref):
    v = tab_ref.shape[0]
    tab = tab_ref[...]
    th = tab.astype(jnp.bfloat16)
    tl = (tab - th.astype(jnp.float32)).astype(jnp.bfloat16)
    z = idx_ref[0, 0, :]
    oht = (
        lax.broadcasted_iota(jnp.int32, (v, _B_TC), 0) == z[None, :]
    ).astype(jnp.bfloat16)
    acc = lax.dot_general(
        oht, th, (((0,), (0,)), ((), ())),
        preferred_element_type=jnp.float32,
    )
    acc = acc + lax.dot_general(
        oht, tl, (((0,), (0,)), ((), ())),
        preferred_element_type=jnp.float32,
    )
    out_ref[...] = acc


def kernel(atomic_numbers, table):
    n = atomic_numbers.shape[0]
    dim = table.shape[1]
    v = table.shape[0]
    n_sc = _M_SC * _K
    n_tc = n - n_sc
    nb_tc = n_tc // _B_TC

    sc_out = _sc_gather(
        atomic_numbers[n_tc:].reshape(_M_SC, _K), table
    ).reshape(n_sc, dim)

    idx3 = atomic_numbers[:n_tc].reshape(nb_tc, 1, _B_TC)
    tc_out = pl_tc.pallas_call(
        _tc_onehot_body,
        grid=(nb_tc,),
        in_specs=[
            pl.BlockSpec((1, 1, _B_TC), lambda i: (i, 0, 0)),
            pl.BlockSpec((v, dim), lambda i: (0, 0)),
        ],
        out_specs=pl.BlockSpec((_B_TC, dim), lambda i: (i, 0)),
        out_shape=jax.ShapeDtypeStruct((n, dim), jnp.float32),
    )(idx3, table)

    return lax.dynamic_update_slice(tc_out, sc_out, (n_tc, 0))


# speedup vs baseline: 1.9885x; 1.2227x over previous
import functools

import jax
from jax import lax
import jax.numpy as jnp
from jax.experimental import pallas as pl
from jax.experimental import pallas as pl_tc
from jax.experimental.pallas import tpu as pltpu
from jax.experimental.pallas import tpu_sc as plsc

_K = 200
_NW = 32
_NBUF = 3
_M_SC = 100
_B_TC = 1600


def _sc_gather(idx2d, table):
    m = idx2d.shape[0]
    dim = table.shape[1]
    niter = (m + _NW - 1) // _NW
    mesh = plsc.VectorSubcoreMesh(core_axis_name="c", subcore_axis_name="s")

    scratch = (
        [pltpu.VMEM((_K,), jnp.int32) for _ in range(_NBUF)]
        + [pltpu.VMEM((_K, dim), table.dtype) for _ in range(_NBUF)]
        + [pltpu.SemaphoreType.DMA for _ in range(3 * _NBUF)]
    )

    @functools.partial(
        pl.kernel,
        out_type=jax.ShapeDtypeStruct((m, _K, dim), table.dtype),
        mesh=mesh,
        scratch_types=scratch,
    )
    def gather_kernel(table_hbm, idx_hbm, out_hbm, *sc):
        idx_v = sc[0:_NBUF]
        row_v = sc[_NBUF:2 * _NBUF]
        sem_i = sc[2 * _NBUF:3 * _NBUF]
        sem_g = sc[3 * _NBUF:4 * _NBUF]
        sem_w = sc[4 * _NBUF:5 * _NBUF]

        wid = lax.axis_index("s") * 2 + lax.axis_index("c")

        def chunk(t):
            return wid + t * _NW

        def start_idx(t, b):
            pltpu.async_copy(idx_hbm.at[chunk(t)], idx_v[b], sem_i[b])

        def wait_idx(b):
            pltpu.make_async_copy(idx_hbm.at[0], idx_v[b], sem_i[b]).wait()

        def start_g(b):
            pltpu.async_copy(table_hbm.at[idx_v[b]], row_v[b], sem_g[b])

        def wait_g(b):
            pltpu.make_async_copy(
                table_hbm.at[idx_v[b]], row_v[b], sem_g[b]
            ).wait()

        def start_w(t, b):
            pltpu.async_copy(row_v[b], out_hbm.at[chunk(t)], sem_w[b])

        def wait_w(b):
            pltpu.make_async_copy(row_v[b], out_hbm.at[0], sem_w[b]).wait()

        for t in range(min(_NBUF, niter)):
            if t == niter - 1:
                pl.when(chunk(t) < m)(lambda t=t: start_idx(t, t % _NBUF))
            else:
                start_idx(t, t % _NBUF)

        for t in range(niter):
            b = t % _NBUF
            if t >= _NBUF:
                wait_w(b)
            if t == niter - 1:
                def last_a(b=b, t=t):
                    wait_idx(b)
                    start_g(b)
                pl.when(chunk(t) < m)(last_a)
            else:
                wait_idx(b)
                start_g(b)
            if t >= 1:
                u = t - 1
                pb = u % _NBUF
                wait_g(pb)
                start_w(u, pb)
                if u + _NBUF < niter:
                    tn = u + _NBUF
                    if tn == niter - 1:
                        pl.when(chunk(tn) < m)(
                            lambda tn=tn, pb=pb: start_idx(tn, pb)
                        )
                    else:
                        start_idx(tn, pb)

        lb = (niter - 1) % _NBUF

        def last_b(lb=lb):
            wait_g(lb)
            start_w(niter - 1, lb)
            wait_w(lb)

        pl.when(chunk(niter - 1) < m)(last_b)
        for t in range(max(0, niter - _NBUF), niter - 1):
            wait_w(t % _NBUF)

    return gather_kernel(table, idx2d)


def _tc_onehot_body(idx_ref, tab_ref, out_ref):
    v, d = tab_ref.shape
    tab = tab_ref[...]
    th = tab.astype(jnp.bfloat16)
    tl = (tab - th.astype(jnp.float32)).astype(jnp.bfloat16)
    tab2 = jnp.concatenate([th, tl], axis=1)
    z = idx_ref[0, 0, :]
    oht = (
        lax.broadcasted_iota(jnp.int32, (v, _B_TC), 0) == z[None, :]
    ).astype(jnp.bfloat16)
    acc = lax.dot_general(
        oht, tab2, (((0,), (0,)), ((), ())),
        preferred_element_type=jnp.float32,
    )
    out_ref[...] = acc[:, :d] + acc[:, d:]


def kernel(atomic_numbers, table):
    n = atomic_numbers.shape[0]
    dim = table.shape[1]
    v = table.shape[0]
    n_sc = _M_SC * _K
    n_tc = n - n_sc
    nb_tc = n_tc // _B_TC

    sc_out = _sc_gather(
        atomic_numbers[n_tc:].reshape(_M_SC, _K), table
    ).reshape(n_sc, dim)

    idx3 = atomic_numbers[:n_tc].reshape(nb_tc, 1, _B_TC)
    tc_out = pl_tc.pallas_call(
        _tc_onehot_body,
        grid=(nb_tc,),
        in_specs=[
            pl.BlockSpec((1, 1, _B_TC), lambda i: (i, 0, 0)),
            pl.BlockSpec((v, dim), lambda i: (0, 0)),
        ],
        out_specs=pl.BlockSpec((_B_TC, dim), lambda i: (i, 0)),
        out_shape=jax.ShapeDtypeStruct((n, dim), jnp.float32),
    )(idx3, table)

    return lax.dynamic_update_slice(tc_out, sc_out, (n_tc, 0))


# speedup vs baseline: 2.1702x; 1.0914x over previous
import functools

import jax
from jax import lax
import jax.numpy as jnp
from jax.experimental import pallas as pl
from jax.experimental import pallas as pl_tc
from jax.experimental.pallas import tpu as pltpu
from jax.experimental.pallas import tpu_sc as plsc

_K = 200
_NW = 32
_NBUF = 3
_M_SC = 116
_B_TC = 2560


def _sc_gather(idx2d, table):
    m = idx2d.shape[0]
    dim = table.shape[1]
    niter = (m + _NW - 1) // _NW
    mesh = plsc.VectorSubcoreMesh(core_axis_name="c", subcore_axis_name="s")

    scratch = (
        [pltpu.VMEM((_K,), jnp.int32) for _ in range(_NBUF)]
        + [pltpu.VMEM((_K, dim), table.dtype) for _ in range(_NBUF)]
        + [pltpu.SemaphoreType.DMA for _ in range(3 * _NBUF)]
    )

    @functools.partial(
        pl.kernel,
        out_type=jax.ShapeDtypeStruct((m, _K, dim), table.dtype),
        mesh=mesh,
        scratch_types=scratch,
    )
    def gather_kernel(table_hbm, idx_hbm, out_hbm, *sc):
        idx_v = sc[0:_NBUF]
        row_v = sc[_NBUF:2 * _NBUF]
        sem_i = sc[2 * _NBUF:3 * _NBUF]
        sem_g = sc[3 * _NBUF:4 * _NBUF]
        sem_w = sc[4 * _NBUF:5 * _NBUF]

        wid = lax.axis_index("s") * 2 + lax.axis_index("c")

        def chunk(t):
            return wid + t * _NW

        def start_idx(t, b):
            pltpu.async_copy(idx_hbm.at[chunk(t)], idx_v[b], sem_i[b])

        def wait_idx(b):
            pltpu.make_async_copy(idx_hbm.at[0], idx_v[b], sem_i[b]).wait()

        def start_g(b):
            pltpu.async_copy(table_hbm.at[idx_v[b]], row_v[b], sem_g[b])

        def wait_g(b):
            pltpu.make_async_copy(
                table_hbm.at[idx_v[b]], row_v[b], sem_g[b]
            ).wait()

        def start_w(t, b):
            pltpu.async_copy(row_v[b], out_hbm.at[chunk(t)], sem_w[b])

        def wait_w(b):
            pltpu.make_async_copy(row_v[b], out_hbm.at[0], sem_w[b]).wait()

        for t in range(min(_NBUF, niter)):
            if t == niter - 1:
                pl.when(chunk(t) < m)(lambda t=t: start_idx(t, t % _NBUF))
            else:
                start_idx(t, t % _NBUF)

        for t in range(niter):
            b = t % _NBUF
            if t >= _NBUF:
                wait_w(b)
            if t == niter - 1:
                def last_a(b=b, t=t):
                    wait_idx(b)
                    start_g(b)
                pl.when(chunk(t) < m)(last_a)
            else:
                wait_idx(b)
                start_g(b)
            if t >= 1:
                u = t - 1
                pb = u % _NBUF
                wait_g(pb)
                start_w(u, pb)
                if u + _NBUF < niter:
                    tn = u + _NBUF
                    if tn == niter - 1:
                        pl.when(chunk(tn) < m)(
                            lambda tn=tn, pb=pb: start_idx(tn, pb)
                        )
                    else:
                        start_idx(tn, pb)

        lb = (niter - 1) % _NBUF

        def last_b(lb=lb):
            wait_g(lb)
            start_w(niter - 1, lb)
            wait_w(lb)

        pl.when(chunk(niter - 1) < m)(last_b)
        for t in range(max(0, niter - _NBUF), niter - 1):
            wait_w(t % _NBUF)

    return gather_kernel(table, idx2d)


def _tc_onehot_body(idx_ref, tab_ref, out_ref):
    v, d = tab_ref.shape
    tab = tab_ref[...]
    th = tab.astype(jnp.bfloat16)
    tl = (tab - th.astype(jnp.float32)).astype(jnp.bfloat16)
    tab2 = jnp.concatenate([th, tl], axis=1)
    z = idx_ref[0, 0, :]
    oht = (
        lax.broadcasted_iota(jnp.int32, (v, _B_TC), 0) == z[None, :]
    ).astype(jnp.bfloat16)
    acc = lax.dot_general(
        oht, tab2, (((0,), (0,)), ((), ())),
        preferred_element_type=jnp.float32,
    )
    out_ref[...] = acc[:, :d] + acc[:, d:]


def kernel(atomic_numbers, table):
    n = atomic_numbers.shape[0]
    dim = table.shape[1]
    v = table.shape[0]
    n_sc = _M_SC * _K
    n_tc = n - n_sc
    nb_tc = n_tc // _B_TC

    sc_out = _sc_gather(
        atomic_numbers[n_tc:].reshape(_M_SC, _K), table
    ).reshape(n_sc, dim)

    idx3 = atomic_numbers[:n_tc].reshape(nb_tc, 1, _B_TC)
    tc_out = pl_tc.pallas_call(
        _tc_onehot_body,
        grid=(nb_tc,),
        in_specs=[
            pl.BlockSpec((1, 1, _B_TC), lambda i: (i, 0, 0)),
            pl.BlockSpec((v, dim), lambda i: (0, 0)),
        ],
        out_specs=pl.BlockSpec((_B_TC, dim), lambda i: (i, 0)),
        out_shape=jax.ShapeDtypeStruct((n, dim), jnp.float32),
    )(idx3, table)

    return lax.dynamic_update_slice(tc_out, sc_out, (n_tc, 0))


# speedup vs baseline: 2.1725x; 1.0010x over previous
import functools

import jax
from jax import lax
import jax.numpy as jnp
from jax.experimental import pallas as pl
from jax.experimental import pallas as pl_tc
from jax.experimental.pallas import tpu as pltpu
from jax.experimental.pallas import tpu_sc as plsc

_K = 200
_NW = 32
_NBUF = 3
_M_SC = 116
_B_TC = 2560


def _sc_gather(idx2d, table):
    m = idx2d.shape[0]
    dim = table.shape[1]
    niter = (m + _NW - 1) // _NW
    mesh = plsc.VectorSubcoreMesh(core_axis_name="c", subcore_axis_name="s")

    scratch = (
        [pltpu.VMEM((_K,), jnp.int32) for _ in range(_NBUF)]
        + [pltpu.VMEM((_K, dim), table.dtype) for _ in range(_NBUF)]
        + [pltpu.SemaphoreType.DMA for _ in range(3 * _NBUF)]
    )

    @functools.partial(
        pl.kernel,
        out_type=jax.ShapeDtypeStruct((m, _K, dim), table.dtype),
        mesh=mesh,
        scratch_types=scratch,
    )
    def gather_kernel(table_hbm, idx_hbm, out_hbm, *sc):
        idx_v = sc[0:_NBUF]
        row_v = sc[_NBUF:2 * _NBUF]
        sem_i = sc[2 * _NBUF:3 * _NBUF]
        sem_g = sc[3 * _NBUF:4 * _NBUF]
        sem_w = sc[4 * _NBUF:5 * _NBUF]

        wid = lax.axis_index("s") * 2 + lax.axis_index("c")

        def chunk(t):
            return wid + t * _NW

        def start_idx(t, b):
            pltpu.async_copy(idx_hbm.at[chunk(t)], idx_v[b], sem_i[b])

        def wait_idx(b):
            pltpu.make_async_copy(idx_hbm.at[0], idx_v[b], sem_i[b]).wait()

        def start_g(b):
            pltpu.async_copy(table_hbm.at[idx_v[b]], row_v[b], sem_g[b])

        def wait_g(b):
            pltpu.make_async_copy(
                table_hbm.at[idx_v[b]], row_v[b], sem_g[b]
            ).wait()

        def start_w(t, b):
            pltpu.async_copy(row_v[b], out_hbm.at[chunk(t)], sem_w[b])

        def wait_w(b):
            pltpu.make_async_copy(row_v[b], out_hbm.at[0], sem_w[b]).wait()

        for t in range(min(_NBUF, niter)):
            if t == niter - 1:
                pl.when(chunk(t) < m)(lambda t=t: start_idx(t, t % _NBUF))
            else:
                start_idx(t, t % _NBUF)

        for t in range(niter):
            b = t % _NBUF
            if t >= _NBUF:
                wait_w(b)
            if t == niter - 1:
                def last_a(b=b, t=t):
                    wait_idx(b)
                    start_g(b)
                pl.when(chunk(t) < m)(last_a)
            else:
                wait_idx(b)
                start_g(b)
            if t >= 1:
                u = t - 1
                pb = u % _NBUF
                wait_g(pb)
                start_w(u, pb)
                if u + _NBUF < niter:
                    tn = u + _NBUF
                    if tn == niter - 1:
                        pl.when(chunk(tn) < m)(
                            lambda tn=tn, pb=pb: start_idx(tn, pb)
                        )
                    else:
                        start_idx(tn, pb)

        lb = (niter - 1) % _NBUF

        def last_b(lb=lb):
            wait_g(lb)
            start_w(niter - 1, lb)
            wait_w(lb)

        pl.when(chunk(niter - 1) < m)(last_b)
        for t in range(max(0, niter - _NBUF), niter - 1):
            wait_w(t % _NBUF)

    return gather_kernel(table, idx2d)


def _tc_onehot_body(idx_ref, tab_ref, out_ref):
    v, d = tab_ref.shape
    tab = tab_ref[...]
    th = tab.astype(jnp.bfloat16)
    tl = (tab - th.astype(jnp.float32)).astype(jnp.bfloat16)
    tab2 = jnp.concatenate([th, tl], axis=1)
    z = idx_ref[0, 0, :]
    oh = (
        z[:, None] == lax.broadcasted_iota(jnp.int32, (_B_TC, v), 1)
    ).astype(jnp.bfloat16)
    acc = jnp.dot(oh, tab2, preferred_element_type=jnp.float32)
    out_ref[...] = acc[:, :d] + acc[:, d:]


def kernel(atomic_numbers, table):
    n = atomic_numbers.shape[0]
    dim = table.shape[1]
    v = table.shape[0]
    n_sc = _M_SC * _K
    n_tc = n - n_sc
    nb_tc = n_tc // _B_TC

    sc_out = _sc_gather(
        atomic_numbers[n_tc:].reshape(_M_SC, _K), table
    ).reshape(n_sc, dim)

    idx3 = atomic_numbers[:n_tc].reshape(nb_tc, 1, _B_TC)
    tc_out = pl_tc.pallas_call(
        _tc_onehot_body,
        grid=(nb_tc,),
        in_specs=[
            pl.BlockSpec((1, 1, _B_TC), lambda i: (i, 0, 0)),
            pl.BlockSpec((v, dim), lambda i: (0, 0)),
        ],
        out_specs=pl.BlockSpec((_B_TC, dim), lambda i: (i, 0)),
        out_shape=jax.ShapeDtypeStruct((n, dim), jnp.float32),
    )(idx3, table)

    return lax.dynamic_update_slice(tc_out, sc_out, (n_tc, 0))


# speedup vs baseline: 2.1730x; 1.0002x over previous
import functools

import jax
from jax import lax
import jax.numpy as jnp
from jax.experimental import pallas as pl
from jax.experimental import pallas as pl_tc
from jax.experimental.pallas import tpu as pltpu
from jax.experimental.pallas import tpu_sc as plsc

_K = 200
_NW = 32
_NBUF = 3
_M_SC = 116
_B_TC = 2560


def _sc_gather(idx2d, table):
    m = idx2d.shape[0]
    dim = table.shape[1]
    niter = (m + _NW - 1) // _NW
    mesh = plsc.VectorSubcoreMesh(core_axis_name="c", subcore_axis_name="s")

    scratch = (
        [pltpu.VMEM((_K,), jnp.int32) for _ in range(_NBUF)]
        + [pltpu.VMEM((_K, dim), table.dtype) for _ in range(_NBUF)]
        + [pltpu.SemaphoreType.DMA for _ in range(3 * _NBUF)]
    )

    @functools.partial(
        pl.kernel,
        out_type=jax.ShapeDtypeStruct((m, _K, dim), table.dtype),
        mesh=mesh,
        scratch_types=scratch,
    )
    def gather_kernel(table_hbm, idx_hbm, out_hbm, *sc):
        idx_v = sc[0:_NBUF]
        row_v = sc[_NBUF:2 * _NBUF]
        sem_i = sc[2 * _NBUF:3 * _NBUF]
        sem_g = sc[3 * _NBUF:4 * _NBUF]
        sem_w = sc[4 * _NBUF:5 * _NBUF]

        wid = lax.axis_index("s") * 2 + lax.axis_index("c")

        def chunk(t):
            return wid + t * _NW

        def start_idx(t, b):
            pltpu.async_copy(idx_hbm.at[chunk(t)], idx_v[b], sem_i[b])

        def wait_idx(b):
            pltpu.make_async_copy(idx_hbm.at[0], idx_v[b], sem_i[b]).wait()

        def start_g(b):
            pltpu.async_copy(table_hbm.at[idx_v[b]], row_v[b], sem_g[b])

        def wait_g(b):
            pltpu.make_async_copy(
                table_hbm.at[idx_v[b]], row_v[b], sem_g[b]
            ).wait()

        def start_w(t, b):
            pltpu.async_copy(row_v[b], out_hbm.at[chunk(t)], sem_w[b])

        def wait_w(b):
            pltpu.make_async_copy(row_v[b], out_hbm.at[0], sem_w[b]).wait()

        for t in range(min(_NBUF, niter)):
            if t == niter - 1:
                pl.when(chunk(t) < m)(lambda t=t: start_idx(t, t % _NBUF))
            else:
                start_idx(t, t % _NBUF)

        for t in range(niter):
            b = t % _NBUF
            if t >= _NBUF:
                wait_w(b)
            if t == niter - 1:
                def last_a(b=b, t=t):
                    wait_idx(b)
                    start_g(b)
                pl.when(chunk(t) < m)(last_a)
            else:
                wait_idx(b)
                start_g(b)
            if t >= 1:
                u = t - 1
                pb = u % _NBUF
                wait_g(pb)
                start_w(u, pb)
                if u + _NBUF < niter:
                    tn = u + _NBUF
                    if tn == niter - 1:
                        pl.when(chunk(tn) < m)(
                            lambda tn=tn, pb=pb: start_idx(tn, pb)
                        )
                    else:
                        start_idx(tn, pb)

        lb = (niter - 1) % _NBUF

        def last_b(lb=lb):
            wait_g(lb)
            start_w(niter - 1, lb)
            wait_w(lb)

        pl.when(chunk(niter - 1) < m)(last_b)
        for t in range(max(0, niter - _NBUF), niter - 1):
            wait_w(t % _NBUF)

    return gather_kernel(table, idx2d)


def _tc_onehot_body(idx_ref, tab_ref, out_ref):
    v, d = tab_ref.shape
    tab = tab_ref[...]
    th = tab.astype(jnp.bfloat16)
    tl = (tab - th.astype(jnp.float32)).astype(jnp.bfloat16)
    zpad = jnp.zeros((128 - v, d), jnp.bfloat16)
    tab2 = jnp.concatenate([th, zpad, tl, zpad], axis=0)
    z = idx_ref[0, 0, :]
    oh2 = (
        z[:, None]
        == (lax.broadcasted_iota(jnp.int32, (_B_TC, 256), 1) & 127)
    ).astype(jnp.bfloat16)
    out_ref[...] = jnp.dot(oh2, tab2, preferred_element_type=jnp.float32)


def kernel(atomic_numbers, table):
    n = atomic_numbers.shape[0]
    dim = table.shape[1]
    v = table.shape[0]
    n_sc = _M_SC * _K
    n_tc = n - n_sc
    nb_tc = n_tc // _B_TC

    sc_out = _sc_gather(
        atomic_numbers[n_tc:].reshape(_M_SC, _K), table
    ).reshape(n_sc, dim)

    idx3 = atomic_numbers[:n_tc].reshape(nb_tc, 1, _B_TC)
    tc_out = pl_tc.pallas_call(
        _tc_onehot_body,
        grid=(nb_tc,),
        in_specs=[
            pl.BlockSpec((1, 1, _B_TC), lambda i: (i, 0, 0)),
            pl.BlockSpec((v, dim), lambda i: (0, 0)),
        ],
        out_specs=pl.BlockSpec((_B_TC, dim), lambda i: (i, 0)),
        out_shape=jax.ShapeDtypeStruct((n, dim), jnp.float32),
    )(idx3, table)

    return lax.dynamic_update_slice(tc_out, sc_out, (n_tc, 0))


# speedup vs baseline: 2.2276x; 1.0251x over previous
import functools

import jax
from jax import lax
import jax.numpy as jnp
from jax.experimental import pallas as pl
from jax.experimental import pallas as pl_tc
from jax.experimental.pallas import tpu as pltpu
from jax.experimental.pallas import tpu_sc as plsc

_K = 200
_NW = 32
_NBUF = 3
_M_SC = 116
_B_TC = 3840


def _sc_gather(idx2d, table):
    m = idx2d.shape[0]
    dim = table.shape[1]
    niter = (m + _NW - 1) // _NW
    mesh = plsc.VectorSubcoreMesh(core_axis_name="c", subcore_axis_name="s")

    scratch = (
        [pltpu.VMEM((_K,), jnp.int32) for _ in range(_NBUF)]
        + [pltpu.VMEM((_K, dim), table.dtype) for _ in range(_NBUF)]
        + [pltpu.SemaphoreType.DMA for _ in range(3 * _NBUF)]
    )

    @functools.partial(
        pl.kernel,
        out_type=jax.ShapeDtypeStruct((m, _K, dim), table.dtype),
        mesh=mesh,
        scratch_types=scratch,
    )
    def gather_kernel(table_hbm, idx_hbm, out_hbm, *sc):
        idx_v = sc[0:_NBUF]
        row_v = sc[_NBUF:2 * _NBUF]
        sem_i = sc[2 * _NBUF:3 * _NBUF]
        sem_g = sc[3 * _NBUF:4 * _NBUF]
        sem_w = sc[4 * _NBUF:5 * _NBUF]

        wid = lax.axis_index("s") * 2 + lax.axis_index("c")

        def chunk(t):
            return wid + t * _NW

        def start_idx(t, b):
            pltpu.async_copy(idx_hbm.at[chunk(t)], idx_v[b], sem_i[b])

        def wait_idx(b):
            pltpu.make_async_copy(idx_hbm.at[0], idx_v[b], sem_i[b]).wait()

        def start_g(b):
            pltpu.async_copy(table_hbm.at[idx_v[b]], row_v[b], sem_g[b])

        def wait_g(b):
            pltpu.make_async_copy(
                table_hbm.at[idx_v[b]], row_v[b], sem_g[b]
            ).wait()

        def start_w(t, b):
            pltpu.async_copy(row_v[b], out_hbm.at[chunk(t)], sem_w[b])

        def wait_w(b):
            pltpu.make_async_copy(row_v[b], out_hbm.at[0], sem_w[b]).wait()

        for t in range(min(_NBUF, niter)):
            if t == niter - 1:
                pl.when(chunk(t) < m)(lambda t=t: start_idx(t, t % _NBUF))
            else:
                start_idx(t, t % _NBUF)

        for t in range(niter):
            b = t % _NBUF
            if t >= _NBUF:
                wait_w(b)
            if t == niter - 1:
                def last_a(b=b, t=t):
                    wait_idx(b)
                    start_g(b)
                pl.when(chunk(t) < m)(last_a)
            else:
                wait_idx(b)
                start_g(b)
            if t >= 1:
                u = t - 1
                pb = u % _NBUF
                wait_g(pb)
                start_w(u, pb)
                if u + _NBUF < niter:
                    tn = u + _NBUF
                    if tn == niter - 1:
                        pl.when(chunk(tn) < m)(
                            lambda tn=tn, pb=pb: start_idx(tn, pb)
                        )
                    else:
                        start_idx(tn, pb)

        lb = (niter - 1) % _NBUF

        def last_b(lb=lb):
            wait_g(lb)
            start_w(niter - 1, lb)
            wait_w(lb)

        pl.when(chunk(niter - 1) < m)(last_b)
        for t in range(max(0, niter - _NBUF), niter - 1):
            wait_w(t % _NBUF)

    return gather_kernel(table, idx2d)


def _tc_onehot_body(idx_ref, tab_ref, out_ref):
    v, d = tab_ref.shape
    tab = tab_ref[...]
    th = tab.astype(jnp.bfloat16)
    tl = (tab - th.astype(jnp.float32)).astype(jnp.bfloat16)
    tab2 = jnp.concatenate([th, tl], axis=1)
    z = idx_ref[0, 0, :]
    oh = (
        z[:, None] == lax.broadcasted_iota(jnp.int32, (_B_TC, v), 1)
    ).astype(jnp.bfloat16)
    acc = jnp.dot(oh, tab2, preferred_element_type=jnp.float32)
    out_ref[...] = acc[:, :d] + acc[:, d:]


def kernel(atomic_numbers, table):
    n = atomic_numbers.shape[0]
    dim = table.shape[1]
    v = table.shape[0]
    n_sc = _M_SC * _K
    n_tc = n - n_sc
    nb_tc = n_tc // _B_TC

    sc_out = _sc_gather(
        atomic_numbers[n_tc:].reshape(_M_SC, _K), table
    ).reshape(n_sc, dim)

    idx3 = atomic_numbers[:n_tc].reshape(nb_tc, 1, _B_TC)
    tc_out = pl_tc.pallas_call(
        _tc_onehot_body,
        grid=(nb_tc,),
        in_specs=[
            pl.BlockSpec((1, 1, _B_TC), lambda i: (i, 0, 0)),
            pl.BlockSpec((v, dim), lambda i: (0, 0)),
        ],
        out_specs=pl.BlockSpec((_B_TC, dim), lambda i: (i, 0)),
        out_shape=jax.ShapeDtypeStruct((n, dim), jnp.float32),
    )(idx3, table)

    return lax.dynamic_update_slice(tc_out, sc_out, (n_tc, 0))
